# baseline (speedup 1.0000x reference)
import functools

import jax
import jax.numpy as jnp
from jax.experimental import pallas as pl
from jax.experimental.pallas import tpu as pltpu

_ROWS = 65536
_BATCH = 256
_BR = 512
_NB = _ROWS // _BR
_NBUF = 16
_LOOK = 8


def _body(x_ref, o_ref, *rest):
    bufs = rest[:_NBUF]
    lsems = rest[_NBUF:2 * _NBUF]
    ssems = rest[2 * _NBUF:3 * _NBUF]

    def load(i, b):
        return pltpu.make_async_copy(
            x_ref.at[pl.ds((i ^ 2) * _BR, _BR)], bufs[b], lsems[b])

    def store(i, b):
        return pltpu.make_async_copy(
            bufs[b], o_ref.at[pl.ds(i * _BR, _BR)], ssems[b])

    for i in range(_LOOK):
        load(i, i % _NBUF).start()
    for i in range(_NB):
        b = i % _NBUF
        nxt = i + _LOOK
        if nxt < _NB:
            bn = nxt % _NBUF
            if nxt >= _NBUF:
                store(nxt - _NBUF, bn).wait()
            load(nxt, bn).start()
        load(i, b).wait()
        store(i, b).start()
    for i in range(_NB - _NBUF, _NB):
        store(i, i % _NBUF).wait()


@functools.partial(jax.jit, donate_argnums=())
def _tc_ring_swap(x):
    scratch = [pltpu.VMEM((_BR, _BATCH), jnp.float32) for _ in range(_NBUF)]
    scratch += [pltpu.SemaphoreType.DMA for _ in range(2 * _NBUF)]
    return pl.pallas_call(
        _body,
        in_specs=[pl.BlockSpec(memory_space=pl.ANY)],
        out_specs=pl.BlockSpec(memory_space=pl.ANY),
        out_shape=jax.ShapeDtypeStruct((_ROWS, _BATCH), jnp.float32),
        scratch_shapes=scratch,
    )(x)


def kernel(x):
    return _tc_ring_swap(x)


# speedup vs baseline: 1.0155x; 1.0155x over previous
import functools

import jax
import jax.numpy as jnp
from jax.experimental import pallas as pl
from jax.experimental.pallas import tpu as pltpu

_ROWS = 65536
_BATCH = 256
_BR = 1024
_NB = _ROWS // _BR
_NBUF = 12
_LOOK = 6


def _body(x_ref, o_ref, *rest):
    bufs = rest[:_NBUF]
    lsems = rest[_NBUF:2 * _NBUF]
    ssems = rest[2 * _NBUF:3 * _NBUF]

    def load(i, b):
        return pltpu.make_async_copy(
            x_ref.at[pl.ds((i ^ 1) * _BR, _BR)], bufs[b], lsems[b])

    def store(i, b):
        return pltpu.make_async_copy(
            bufs[b], o_ref.at[pl.ds(i * _BR, _BR)], ssems[b])

    for i in range(_LOOK):
        load(i, i % _NBUF).start()
    for i in range(_NB):
        b = i % _NBUF
        nxt = i + _LOOK
        if nxt < _NB:
            bn = nxt % _NBUF
            if nxt >= _NBUF:
                store(nxt - _NBUF, bn).wait()
            load(nxt, bn).start()
        load(i, b).wait()
        store(i, b).start()
    for i in range(_NB - _NBUF, _NB):
        store(i, i % _NBUF).wait()


@functools.partial(jax.jit, donate_argnums=())
def _tc_ring_swap(x):
    scratch = [pltpu.VMEM((_BR, _BATCH), jnp.float32) for _ in range(_NBUF)]
    scratch += [pltpu.SemaphoreType.DMA for _ in range(2 * _NBUF)]
    return pl.pallas_call(
        _body,
        in_specs=[pl.BlockSpec(memory_space=pl.ANY)],
        out_specs=pl.BlockSpec(memory_space=pl.ANY),
        out_shape=jax.ShapeDtypeStruct((_ROWS, _BATCH), jnp.float32),
        scratch_shapes=scratch,
    )(x)


def kernel(x):
    return _tc_ring_swap(x)
